# baseline (speedup 1.0000x reference)
import jax
import jax.numpy as jnp
from jax.experimental import pallas as pl
from jax.experimental.pallas import tpu as pltpu

_EPS = 1e-5
_HW = 3136


def _gct_body(x_ref, a_ref, g_ref, b_ref, o_ref):
    x = x_ref[0]
    xx = x * x
    lanes = jax.lax.broadcasted_iota(jnp.int32, (1, 2 * _HW), 1)
    hi = lanes >= _HW
    s_hi = jnp.sum(jnp.where(hi, xx, 0.0), axis=1, keepdims=True)
    s_lo = jnp.sum(xx, axis=1, keepdims=True) - s_hi
    a = a_ref[...]
    e_lo = jnp.sqrt(s_lo + _EPS) * a[:, 0:1]
    e_hi = jnp.sqrt(s_hi + _EPS) * a[:, 1:2]
    msq = (jnp.sum(e_lo * e_lo) + jnp.sum(e_hi * e_hi)) * (1.0 / 256.0)
    inv = jax.lax.rsqrt(msq + _EPS)
    g = g_ref[...]
    b = b_ref[...]
    gate_lo = 1.0 + jnp.tanh(e_lo * g[:, 0:1] * inv + b[:, 0:1])
    gate_hi = 1.0 + jnp.tanh(e_hi * g[:, 1:2] * inv + b[:, 1:2])
    o_ref[0] = x * jnp.where(hi, gate_hi, gate_lo)


def kernel(x, alpha, gamma, beta):
    N, C, H, W = x.shape
    xr = x.reshape(N, C // 2, 2 * H * W)
    a2 = alpha.reshape(C // 2, 2)
    g2 = gamma.reshape(C // 2, 2)
    b2 = beta.reshape(C // 2, 2)
    blk = (1, C // 2, 2 * H * W)
    out = pl.pallas_call(
        _gct_body,
        grid=(N,),
        in_specs=[
            pl.BlockSpec(blk, lambda n: (n, 0, 0)),
            pl.BlockSpec((C // 2, 2), lambda n: (0, 0)),
            pl.BlockSpec((C // 2, 2), lambda n: (0, 0)),
            pl.BlockSpec((C // 2, 2), lambda n: (0, 0)),
        ],
        out_specs=pl.BlockSpec(blk, lambda n: (n, 0, 0)),
        out_shape=jax.ShapeDtypeStruct((N, C // 2, 2 * H * W), x.dtype),
        compiler_params=pltpu.CompilerParams(
            dimension_semantics=("parallel",)
        ),
    )(xr, a2, g2, b2)
    return out.reshape(N, C, H, W)


# speedup vs baseline: 1.4128x; 1.4128x over previous
import jax
import jax.numpy as jnp
from jax.experimental import pallas as pl
from jax.experimental.pallas import tpu as pltpu

_EPS = 1e-5


def _gct_body(x_ref, a_ref, g_ref, b_ref, o_ref):
    x = x_ref[...]
    sumsq = jnp.sum(x * x, axis=(2, 3))
    embed = jnp.sqrt(sumsq + _EPS) * a_ref[...]
    inv = jax.lax.rsqrt(
        jnp.mean(embed * embed, axis=1, keepdims=True) + _EPS
    )
    z = embed * g_ref[...] * inv + b_ref[...]
    gate = 1.0 + jnp.tanh(z)
    o_ref[...] = x * gate[:, :, None, None]


def kernel(x, alpha, gamma, beta):
    N, C, H, W = x.shape
    a2 = alpha.reshape(1, C)
    g2 = gamma.reshape(1, C)
    b2 = beta.reshape(1, C)
    blk = (1, C, H, W)
    out = pl.pallas_call(
        _gct_body,
        grid=(N,),
        in_specs=[
            pl.BlockSpec(blk, lambda n: (n, 0, 0, 0)),
            pl.BlockSpec((1, C), lambda n: (0, 0)),
            pl.BlockSpec((1, C), lambda n: (0, 0)),
            pl.BlockSpec((1, C), lambda n: (0, 0)),
        ],
        out_specs=pl.BlockSpec(blk, lambda n: (n, 0, 0, 0)),
        out_shape=jax.ShapeDtypeStruct((N, C, H, W), x.dtype),
        compiler_params=pltpu.CompilerParams(
            dimension_semantics=("parallel",)
        ),
    )(x, a2, g2, b2)
    return out
